# baseline (speedup 1.0000x reference)
import functools

import jax
import jax.numpy as jnp
from jax import lax
from jax.experimental import pallas as pl
from jax.experimental.pallas import tpu as pltpu
from jax.experimental.pallas import tpu_sc as plsc

_N = 10000
_E = 320000
_D = 128
_DH = 64
_NS = 16
_NC = 2
_BLK = 128
_NIT = 157
_EPS = _NIT * _BLK
_EPAD = _EPS * _NS
_AROWS = 10112
_RPS = _AROWS // _NS
_CW = 16
_NBUF = 4
_R = 2000
_NB = _N // _R

_sc_mesh = plsc.VectorSubcoreMesh(core_axis_name="c", subcore_axis_name="s")


def _seg_body(with_counts, *refs):
    if with_counts:
        (x, srcs, dsts, z_d, z_c, ones_c, sum_out, cnt_out,
         idx_s, idx_d, rows, acc, ones_v, cacc, sem, ssem, csem) = refs
    else:
        (x, srcs, dsts, z_d, sum_out,
         idx_s, idx_d, rows, acc, sem, ssem) = refs
        csem = None
    c = lax.axis_index("c")
    s = lax.axis_index("s")
    pltpu.sync_copy(z_d, acc.at[pl.ds(s * _RPS, _RPS)])
    pltpu.sync_copy(srcs.at[c, s], idx_s)
    pltpu.sync_copy(dsts.at[c, s], idx_d)
    if with_counts:
        pltpu.sync_copy(z_c, cacc.at[pl.ds(s * _RPS, _RPS)])
        pltpu.sync_copy(ones_c, ones_v)
    plsc.subcore_barrier()

    for p in range(_NBUF - 1):
        pltpu.async_copy(x.at[idx_s.at[p]], rows.at[p], sem)

    def step(j, carry):
        b = j % _NBUF
        pltpu.make_async_copy(x.at[idx_s.at[j]], rows.at[b], sem).wait()
        pltpu.async_copy(rows.at[b], acc.at[idx_d.at[j]], ssem, add=True)
        if with_counts:
            pltpu.async_copy(ones_v, cacc.at[idx_d.at[j]], csem, add=True)

        @pl.when(j >= 1)
        def _():
            bp = (j - 1) % _NBUF
            pltpu.make_async_copy(rows.at[bp], acc.at[idx_d.at[j - 1]],
                                  ssem).wait()
            if with_counts:
                pltpu.make_async_copy(ones_v, cacc.at[idx_d.at[j - 1]],
                                      csem).wait()

        @pl.when(j + _NBUF - 1 < _NIT)
        def _():
            pltpu.async_copy(x.at[idx_s.at[j + _NBUF - 1]],
                             rows.at[(j + _NBUF - 1) % _NBUF], sem)
        return carry

    lax.fori_loop(0, _NIT, step, 0)
    pltpu.make_async_copy(rows.at[(_NIT - 1) % _NBUF],
                          acc.at[idx_d.at[_NIT - 1]], ssem).wait()
    if with_counts:
        pltpu.make_async_copy(ones_v, cacc.at[idx_d.at[_NIT - 1]],
                              csem).wait()
    plsc.subcore_barrier()
    pltpu.sync_copy(acc.at[pl.ds(s * _RPS, _RPS)],
                    sum_out.at[c, pl.ds(s * _RPS, _RPS)])
    if with_counts:
        pltpu.sync_copy(cacc.at[pl.ds(s * _RPS, _RPS)],
                        cnt_out.at[c, pl.ds(s * _RPS, _RPS)])


def _make_seg_sum(with_counts):
    out_type = jax.ShapeDtypeStruct((_NC, _AROWS, _DH), jnp.float32)
    if with_counts:
        out_type = (out_type,
                    jax.ShapeDtypeStruct((_NC, _AROWS, _CW), jnp.float32))
    scratch = [
        pltpu.VMEM((_NIT, _BLK), jnp.int32),
        pltpu.VMEM((_NIT, _BLK), jnp.int32),
        pltpu.VMEM((_NBUF, _BLK, _DH), jnp.float32),
        pltpu.VMEM_SHARED((_AROWS, _DH), jnp.float32),
    ]
    if with_counts:
        scratch += [
            pltpu.VMEM((_BLK, _CW), jnp.float32),
            pltpu.VMEM_SHARED((_AROWS, _CW), jnp.float32),
        ]
    scratch.append(pltpu.SemaphoreType.DMA)
    scratch.append(pltpu.SemaphoreType.DMA)
    if with_counts:
        scratch.append(pltpu.SemaphoreType.DMA)
    return pl.kernel(
        functools.partial(_seg_body, with_counts),
        out_type=out_type,
        mesh=_sc_mesh,
        scratch_types=scratch,
        compiler_params=pltpu.CompilerParams(use_tc_tiling_on_sc=False),
    )


def _proj_body(x, w, b, out_lo, out_hi):
    y = jnp.maximum(
        jnp.dot(x[...], w[0], preferred_element_type=jnp.float32,
                precision=lax.Precision.HIGHEST) + b[0], 0.0)
    out_lo[...] = y[:, :_DH]
    out_hi[...] = y[:, _DH:]


def _proj(x, w, b):
    return pl.pallas_call(
        _proj_body,
        grid=(2, _NB),
        in_specs=[
            pl.BlockSpec((_R, _D), lambda t, r: (t * _NB + r, 0)),
            pl.BlockSpec((1, _D, _D), lambda t, r: (t, 0, 0)),
            pl.BlockSpec((1, 1, _D), lambda t, r: (t, 0, 0)),
        ],
        out_specs=[
            pl.BlockSpec((_R, _DH), lambda t, r: (t * _NB + r, 0)),
            pl.BlockSpec((_R, _DH), lambda t, r: (t * _NB + r, 0)),
        ],
        out_shape=[
            jax.ShapeDtypeStruct((2 * _N, _DH), jnp.float32),
            jax.ShapeDtypeStruct((2 * _N, _DH), jnp.float32),
        ],
    )(x, w, b)


def _layer_body(sums_lo, sums_hi, cnts, x_lo, x_hi, wn, bn, ws, bs, g, b,
                out_lo, out_hi):
    seg = jnp.concatenate([sums_lo[0], sums_hi[0]], axis=-1)
    x = jnp.concatenate([x_lo[...], x_hi[...]], axis=-1)
    cnt = cnts[0, :, 0:1]
    mean = seg / jnp.maximum(cnt, 1.0)
    agg = jnp.dot(mean, wn[0], preferred_element_type=jnp.float32,
                precision=lax.Precision.HIGHEST)
    agg = agg + jnp.where(cnt > 0.0, 1.0, 0.0) * bn[0]
    res = agg + jnp.dot(x, ws[0], preferred_element_type=jnp.float32,
                precision=lax.Precision.HIGHEST) + bs[0]
    mu = jnp.mean(res, axis=-1, keepdims=True)
    var = jnp.mean((res - mu) ** 2, axis=-1, keepdims=True)
    y = (res - mu) / jnp.sqrt(var + 1e-5) * g[0] + b[0]
    y = jnp.maximum(y, 0.0)
    out_lo[...] = y[:, :_DH]
    out_hi[...] = y[:, _DH:]


def _layer(sums_lo, sums_hi, cnts, x_lo, x_hi, wn, bn, ws, bs, g, b):
    return pl.pallas_call(
        _layer_body,
        grid=(2, _NB),
        in_specs=[
            pl.BlockSpec((1, _R, _DH), lambda t, r: (t, r, 0)),
            pl.BlockSpec((1, _R, _DH), lambda t, r: (t, r, 0)),
            pl.BlockSpec((1, _R, _CW), lambda t, r: (t, r, 0)),
            pl.BlockSpec((_R, _DH), lambda t, r: (t * _NB + r, 0)),
            pl.BlockSpec((_R, _DH), lambda t, r: (t * _NB + r, 0)),
            pl.BlockSpec((1, _D, _D), lambda t, r: (t, 0, 0)),
            pl.BlockSpec((1, 1, _D), lambda t, r: (t, 0, 0)),
            pl.BlockSpec((1, _D, _D), lambda t, r: (t, 0, 0)),
            pl.BlockSpec((1, 1, _D), lambda t, r: (t, 0, 0)),
            pl.BlockSpec((1, 1, _D), lambda t, r: (t, 0, 0)),
            pl.BlockSpec((1, 1, _D), lambda t, r: (t, 0, 0)),
        ],
        out_specs=[
            pl.BlockSpec((_R, _DH), lambda t, r: (t * _NB + r, 0)),
            pl.BlockSpec((_R, _DH), lambda t, r: (t * _NB + r, 0)),
        ],
        out_shape=[
            jax.ShapeDtypeStruct((2 * _N, _DH), jnp.float32),
            jax.ShapeDtypeStruct((2 * _N, _DH), jnp.float32),
        ],
    )(sums_lo, sums_hi, cnts, x_lo, x_hi, wn, bn, ws, bs, g, b)


def _head_body(x_lo, x_hi, w1, b1, w2, b2, out):
    x = jnp.concatenate([x_lo[...], x_hi[...]], axis=-1)
    h = jnp.maximum(
        jnp.dot(x, w1[...], preferred_element_type=jnp.float32,
                precision=lax.Precision.HIGHEST) + b1[0], 0.0)
    res = jnp.dot(h, w2[...], preferred_element_type=jnp.float32,
                precision=lax.Precision.HIGHEST) + b2[0, 0, 0]
    out[...] = res[:, 0]


def _head(x_lo, x_hi, w1, b1, w2, b2):
    return pl.pallas_call(
        _head_body,
        grid=(1,),
        in_specs=[
            pl.BlockSpec((_N, _DH), lambda i: (0, 0)),
            pl.BlockSpec((_N, _DH), lambda i: (0, 0)),
            pl.BlockSpec((_D, _D // 2), lambda i: (0, 0)),
            pl.BlockSpec((1, 1, _D // 2), lambda i: (0, 0, 0)),
            pl.BlockSpec((_D // 2, 1), lambda i: (0, 0)),
            pl.BlockSpec((1, 1, 1), lambda i: (0, 0, 0)),
        ],
        out_specs=pl.BlockSpec((_N,), lambda i: (0,)),
        out_shape=jax.ShapeDtypeStruct((_N,), jnp.float32),
    )(x_lo, x_hi, w1, b1, w2, b2)


def _stack_w(a, b):
    return jnp.stack([a, b])


def _stack_b(a, b):
    return jnp.stack([a, b]).reshape(2, 1, _D)


def kernel(x_AdsInfo, x_User, e_user_ad_src, e_user_ad_dst, e_ad_user_src,
           e_ad_user_dst, n_AdsInfo, n_User,
           in_W_AdsInfo, in_b_AdsInfo, in_W_User, in_b_User,
           Wn_l0_user_to_ad, bn_l0_user_to_ad, Ws_l0_user_to_ad,
           bs_l0_user_to_ad, Wn_l0_ad_to_user, bn_l0_ad_to_user,
           Ws_l0_ad_to_user, bs_l0_ad_to_user, ln_g_l0_AdsInfo,
           ln_b_l0_AdsInfo, ln_g_l0_User, ln_b_l0_User,
           Wn_l1_user_to_ad, bn_l1_user_to_ad, Ws_l1_user_to_ad,
           bs_l1_user_to_ad, Wn_l1_ad_to_user, bn_l1_ad_to_user,
           Ws_l1_ad_to_user, bs_l1_ad_to_user, ln_g_l1_AdsInfo,
           ln_b_l1_AdsInfo, ln_g_l1_User, ln_b_l1_User,
           head_W1, head_b1, head_W2, head_b2):
    pad = _EPAD - _E

    def _pad_idx(a, fill):
        return jnp.concatenate(
            [a.astype(jnp.int32), jnp.full((pad,), fill, jnp.int32)])

    srcs = jnp.stack([
        _pad_idx(e_user_ad_src + _N, 0),
        _pad_idx(e_ad_user_src, 0),
    ]).reshape(_NC, _NS, _NIT, _BLK)
    dsts = jnp.stack([
        _pad_idx(e_user_ad_dst, _N),
        _pad_idx(e_ad_user_dst, _N),
    ]).reshape(_NC, _NS, _NIT, _BLK)

    z_d = jnp.zeros((_RPS, _DH), jnp.float32)
    z_c = jnp.zeros((_RPS, _CW), jnp.float32)
    ones_c = jnp.ones((_BLK, _CW), jnp.float32)

    seg_cnt = _make_seg_sum(True)
    seg = _make_seg_sum(False)

    x_cat = jnp.concatenate([x_AdsInfo, x_User], axis=0)
    x0_lo, x0_hi = _proj(x_cat, _stack_w(in_W_AdsInfo, in_W_User),
                         _stack_b(in_b_AdsInfo, in_b_User))

    sums0_lo, cnts = seg_cnt(x0_lo, srcs, dsts, z_d, z_c, ones_c)
    sums0_hi = seg(x0_hi, srcs, dsts, z_d)
    x1_lo, x1_hi = _layer(sums0_lo, sums0_hi, cnts, x0_lo, x0_hi,
                          _stack_w(Wn_l0_user_to_ad, Wn_l0_ad_to_user),
                          _stack_b(bn_l0_user_to_ad, bn_l0_ad_to_user),
                          _stack_w(Ws_l0_user_to_ad, Ws_l0_ad_to_user),
                          _stack_b(bs_l0_user_to_ad, bs_l0_ad_to_user),
                          _stack_b(ln_g_l0_AdsInfo, ln_g_l0_User),
                          _stack_b(ln_b_l0_AdsInfo, ln_b_l0_User))

    sums1_lo = seg(x1_lo, srcs, dsts, z_d)
    sums1_hi = seg(x1_hi, srcs, dsts, z_d)
    x2_lo, x2_hi = _layer(sums1_lo, sums1_hi, cnts, x1_lo, x1_hi,
                          _stack_w(Wn_l1_user_to_ad, Wn_l1_ad_to_user),
                          _stack_b(bn_l1_user_to_ad, bn_l1_ad_to_user),
                          _stack_w(Ws_l1_user_to_ad, Ws_l1_ad_to_user),
                          _stack_b(bs_l1_user_to_ad, bs_l1_ad_to_user),
                          _stack_b(ln_g_l1_AdsInfo, ln_g_l1_User),
                          _stack_b(ln_b_l1_AdsInfo, ln_b_l1_User))

    return _head(x2_lo, x2_hi, head_W1, head_b1.reshape(1, 1, _D // 2),
                 head_W2, head_b2.reshape(1, 1, 1))


# speedup vs baseline: 1.0858x; 1.0858x over previous
import functools

import jax
import jax.numpy as jnp
from jax import lax
from jax.experimental import pallas as pl
from jax.experimental.pallas import tpu as pltpu
from jax.experimental.pallas import tpu_sc as plsc

_N = 10000
_E = 320000
_D = 128
_DH = 64
_NS = 16
_NC = 2
_BLK = 128
_NIT = 157
_EPS = _NIT * _BLK
_EPAD = _EPS * _NS
_AROWS = 10112
_RPS = _AROWS // _NS
_CW = 16
_NBUF = 4
_NBUF_P = 6
_R = 2000
_NB = _N // _R

_sc_mesh = plsc.VectorSubcoreMesh(core_axis_name="c", subcore_axis_name="s")


def _seg_body(with_counts, nbuf, *refs):
    if with_counts:
        (x, srcs, dsts, z_d, z_c, ones_c, sum_out, cnt_out,
         idx_s, idx_d, rows, acc, ones_v, cacc, sem, ssem, csem) = refs
    else:
        (x, srcs, dsts, z_d, sum_out,
         idx_s, idx_d, rows, acc, sem, ssem) = refs
        csem = None
    c = lax.axis_index("c")
    s = lax.axis_index("s")
    pltpu.sync_copy(z_d, acc.at[pl.ds(s * _RPS, _RPS)])
    pltpu.sync_copy(srcs.at[c, s], idx_s)
    pltpu.sync_copy(dsts.at[c, s], idx_d)
    if with_counts:
        pltpu.sync_copy(z_c, cacc.at[pl.ds(s * _RPS, _RPS)])
        pltpu.sync_copy(ones_c, ones_v)
    plsc.subcore_barrier()

    for p in range(nbuf - 1):
        pltpu.async_copy(x.at[idx_s.at[p]], rows.at[p], sem)

    def step(j, carry):
        b = j % nbuf
        pltpu.make_async_copy(x.at[idx_s.at[j]], rows.at[b], sem).wait()
        pltpu.async_copy(rows.at[b], acc.at[idx_d.at[j]], ssem, add=True)
        if with_counts:
            pltpu.async_copy(ones_v, cacc.at[idx_d.at[j]], csem, add=True)

        @pl.when(j >= 1)
        def _():
            bp = (j - 1) % nbuf
            pltpu.make_async_copy(rows.at[bp], acc.at[idx_d.at[j - 1]],
                                  ssem).wait()
            if with_counts:
                pltpu.make_async_copy(ones_v, cacc.at[idx_d.at[j - 1]],
                                      csem).wait()

        @pl.when(j + nbuf - 1 < _NIT)
        def _():
            pltpu.async_copy(x.at[idx_s.at[j + nbuf - 1]],
                             rows.at[(j + nbuf - 1) % nbuf], sem)
        return carry

    lax.fori_loop(0, _NIT, step, 0)
    pltpu.make_async_copy(rows.at[(_NIT - 1) % nbuf],
                          acc.at[idx_d.at[_NIT - 1]], ssem).wait()
    if with_counts:
        pltpu.make_async_copy(ones_v, cacc.at[idx_d.at[_NIT - 1]],
                              csem).wait()
    plsc.subcore_barrier()
    pltpu.sync_copy(acc.at[pl.ds(s * _RPS, _RPS)],
                    sum_out.at[c, pl.ds(s * _RPS, _RPS)])
    if with_counts:
        pltpu.sync_copy(cacc.at[pl.ds(s * _RPS, _RPS)],
                        cnt_out.at[c, pl.ds(s * _RPS, _RPS)])


def _make_seg_sum(with_counts):
    nbuf = _NBUF if with_counts else _NBUF_P
    out_type = jax.ShapeDtypeStruct((_NC, _AROWS, _DH), jnp.float32)
    if with_counts:
        out_type = (out_type,
                    jax.ShapeDtypeStruct((_NC, _AROWS, _CW), jnp.float32))
    scratch = [
        pltpu.VMEM((_NIT, _BLK), jnp.int32),
        pltpu.VMEM((_NIT, _BLK), jnp.int32),
        pltpu.VMEM((nbuf, _BLK, _DH), jnp.float32),
        pltpu.VMEM_SHARED((_AROWS, _DH), jnp.float32),
    ]
    if with_counts:
        scratch += [
            pltpu.VMEM((_BLK, _CW), jnp.float32),
            pltpu.VMEM_SHARED((_AROWS, _CW), jnp.float32),
        ]
    scratch.append(pltpu.SemaphoreType.DMA)
    scratch.append(pltpu.SemaphoreType.DMA)
    if with_counts:
        scratch.append(pltpu.SemaphoreType.DMA)
    return pl.kernel(
        functools.partial(_seg_body, with_counts, nbuf),
        out_type=out_type,
        mesh=_sc_mesh,
        scratch_types=scratch,
        compiler_params=pltpu.CompilerParams(use_tc_tiling_on_sc=False),
    )


def _proj_body(x, w, b, out_lo, out_hi):
    y = jnp.maximum(
        jnp.dot(x[...], w[0], preferred_element_type=jnp.float32) + b[0], 0.0)
    out_lo[...] = y[:, :_DH]
    out_hi[...] = y[:, _DH:]


def _proj(x, w, b):
    return pl.pallas_call(
        _proj_body,
        grid=(2, _NB),
        in_specs=[
            pl.BlockSpec((_R, _D), lambda t, r: (t * _NB + r, 0)),
            pl.BlockSpec((1, _D, _D), lambda t, r: (t, 0, 0)),
            pl.BlockSpec((1, 1, _D), lambda t, r: (t, 0, 0)),
        ],
        out_specs=[
            pl.BlockSpec((_R, _DH), lambda t, r: (t * _NB + r, 0)),
            pl.BlockSpec((_R, _DH), lambda t, r: (t * _NB + r, 0)),
        ],
        out_shape=[
            jax.ShapeDtypeStruct((2 * _N, _DH), jnp.float32),
            jax.ShapeDtypeStruct((2 * _N, _DH), jnp.float32),
        ],
    )(x, w, b)


def _layer_body(sums_lo, sums_hi, cnts, x_lo, x_hi, wn, bn, ws, bs, g, b,
                out_lo, out_hi):
    seg = jnp.concatenate([sums_lo[0], sums_hi[0]], axis=-1)
    x = jnp.concatenate([x_lo[...], x_hi[...]], axis=-1)
    cnt = cnts[0, :, 0:1]
    mean = seg / jnp.maximum(cnt, 1.0)
    agg = jnp.dot(mean, wn[0], preferred_element_type=jnp.float32)
    agg = agg + jnp.where(cnt > 0.0, 1.0, 0.0) * bn[0]
    res = agg + jnp.dot(x, ws[0], preferred_element_type=jnp.float32) + bs[0]
    mu = jnp.mean(res, axis=-1, keepdims=True)
    var = jnp.mean((res - mu) ** 2, axis=-1, keepdims=True)
    y = (res - mu) / jnp.sqrt(var + 1e-5) * g[0] + b[0]
    y = jnp.maximum(y, 0.0)
    out_lo[...] = y[:, :_DH]
    out_hi[...] = y[:, _DH:]


def _layer(sums_lo, sums_hi, cnts, x_lo, x_hi, wn, bn, ws, bs, g, b):
    return pl.pallas_call(
        _layer_body,
        grid=(2, _NB),
        in_specs=[
            pl.BlockSpec((1, _R, _DH), lambda t, r: (t, r, 0)),
            pl.BlockSpec((1, _R, _DH), lambda t, r: (t, r, 0)),
            pl.BlockSpec((1, _R, _CW), lambda t, r: (t, r, 0)),
            pl.BlockSpec((_R, _DH), lambda t, r: (t * _NB + r, 0)),
            pl.BlockSpec((_R, _DH), lambda t, r: (t * _NB + r, 0)),
            pl.BlockSpec((1, _D, _D), lambda t, r: (t, 0, 0)),
            pl.BlockSpec((1, 1, _D), lambda t, r: (t, 0, 0)),
            pl.BlockSpec((1, _D, _D), lambda t, r: (t, 0, 0)),
            pl.BlockSpec((1, 1, _D), lambda t, r: (t, 0, 0)),
            pl.BlockSpec((1, 1, _D), lambda t, r: (t, 0, 0)),
            pl.BlockSpec((1, 1, _D), lambda t, r: (t, 0, 0)),
        ],
        out_specs=[
            pl.BlockSpec((_R, _DH), lambda t, r: (t * _NB + r, 0)),
            pl.BlockSpec((_R, _DH), lambda t, r: (t * _NB + r, 0)),
        ],
        out_shape=[
            jax.ShapeDtypeStruct((2 * _N, _DH), jnp.float32),
            jax.ShapeDtypeStruct((2 * _N, _DH), jnp.float32),
        ],
    )(sums_lo, sums_hi, cnts, x_lo, x_hi, wn, bn, ws, bs, g, b)


def _head_body(x_lo, x_hi, w1, b1, w2, b2, out):
    x = jnp.concatenate([x_lo[...], x_hi[...]], axis=-1)
    h = jnp.maximum(
        jnp.dot(x, w1[...], preferred_element_type=jnp.float32) + b1[0], 0.0)
    res = jnp.dot(h, w2[...], preferred_element_type=jnp.float32) + b2[0, 0, 0]
    out[...] = res[:, 0]


def _head(x_lo, x_hi, w1, b1, w2, b2):
    return pl.pallas_call(
        _head_body,
        grid=(1,),
        in_specs=[
            pl.BlockSpec((_N, _DH), lambda i: (0, 0)),
            pl.BlockSpec((_N, _DH), lambda i: (0, 0)),
            pl.BlockSpec((_D, _D // 2), lambda i: (0, 0)),
            pl.BlockSpec((1, 1, _D // 2), lambda i: (0, 0, 0)),
            pl.BlockSpec((_D // 2, 1), lambda i: (0, 0)),
            pl.BlockSpec((1, 1, 1), lambda i: (0, 0, 0)),
        ],
        out_specs=pl.BlockSpec((_N,), lambda i: (0,)),
        out_shape=jax.ShapeDtypeStruct((_N,), jnp.float32),
    )(x_lo, x_hi, w1, b1, w2, b2)


def _stack_w(a, b):
    return jnp.stack([a, b])


def _stack_b(a, b):
    return jnp.stack([a, b]).reshape(2, 1, _D)


def kernel(x_AdsInfo, x_User, e_user_ad_src, e_user_ad_dst, e_ad_user_src,
           e_ad_user_dst, n_AdsInfo, n_User,
           in_W_AdsInfo, in_b_AdsInfo, in_W_User, in_b_User,
           Wn_l0_user_to_ad, bn_l0_user_to_ad, Ws_l0_user_to_ad,
           bs_l0_user_to_ad, Wn_l0_ad_to_user, bn_l0_ad_to_user,
           Ws_l0_ad_to_user, bs_l0_ad_to_user, ln_g_l0_AdsInfo,
           ln_b_l0_AdsInfo, ln_g_l0_User, ln_b_l0_User,
           Wn_l1_user_to_ad, bn_l1_user_to_ad, Ws_l1_user_to_ad,
           bs_l1_user_to_ad, Wn_l1_ad_to_user, bn_l1_ad_to_user,
           Ws_l1_ad_to_user, bs_l1_ad_to_user, ln_g_l1_AdsInfo,
           ln_b_l1_AdsInfo, ln_g_l1_User, ln_b_l1_User,
           head_W1, head_b1, head_W2, head_b2):
    pad = _EPAD - _E

    def _pad_idx(a, fill):
        return jnp.concatenate(
            [a.astype(jnp.int32), jnp.full((pad,), fill, jnp.int32)])

    srcs = jnp.stack([
        _pad_idx(e_user_ad_src + _N, 0),
        _pad_idx(e_ad_user_src, 0),
    ]).reshape(_NC, _NS, _NIT, _BLK)
    dsts = jnp.stack([
        _pad_idx(e_user_ad_dst, _N),
        _pad_idx(e_ad_user_dst, _N),
    ]).reshape(_NC, _NS, _NIT, _BLK)

    z_d = jnp.zeros((_RPS, _DH), jnp.float32)
    z_c = jnp.zeros((_RPS, _CW), jnp.float32)
    ones_c = jnp.ones((_BLK, _CW), jnp.float32)

    seg_cnt = _make_seg_sum(True)
    seg = _make_seg_sum(False)

    x_cat = jnp.concatenate([x_AdsInfo, x_User], axis=0)
    x0_lo, x0_hi = _proj(x_cat, _stack_w(in_W_AdsInfo, in_W_User),
                         _stack_b(in_b_AdsInfo, in_b_User))

    sums0_lo, cnts = seg_cnt(x0_lo, srcs, dsts, z_d, z_c, ones_c)
    sums0_hi = seg(x0_hi, srcs, dsts, z_d)
    x1_lo, x1_hi = _layer(sums0_lo, sums0_hi, cnts, x0_lo, x0_hi,
                          _stack_w(Wn_l0_user_to_ad, Wn_l0_ad_to_user),
                          _stack_b(bn_l0_user_to_ad, bn_l0_ad_to_user),
                          _stack_w(Ws_l0_user_to_ad, Ws_l0_ad_to_user),
                          _stack_b(bs_l0_user_to_ad, bs_l0_ad_to_user),
                          _stack_b(ln_g_l0_AdsInfo, ln_g_l0_User),
                          _stack_b(ln_b_l0_AdsInfo, ln_b_l0_User))

    sums1_lo = seg(x1_lo, srcs, dsts, z_d)
    sums1_hi = seg(x1_hi, srcs, dsts, z_d)
    x2_lo, x2_hi = _layer(sums1_lo, sums1_hi, cnts, x1_lo, x1_hi,
                          _stack_w(Wn_l1_user_to_ad, Wn_l1_ad_to_user),
                          _stack_b(bn_l1_user_to_ad, bn_l1_ad_to_user),
                          _stack_w(Ws_l1_user_to_ad, Ws_l1_ad_to_user),
                          _stack_b(bs_l1_user_to_ad, bs_l1_ad_to_user),
                          _stack_b(ln_g_l1_AdsInfo, ln_g_l1_User),
                          _stack_b(ln_b_l1_AdsInfo, ln_b_l1_User))

    return _head(x2_lo, x2_hi, head_W1, head_b1.reshape(1, 1, _D // 2),
                 head_W2, head_b2.reshape(1, 1, 1))
